# Initial kernel scaffold; baseline (speedup 1.0000x reference)
import jax
import jax.numpy as jnp
from jax.experimental import pallas as pl


def kernel(x, token_table, pos_table):
    raise NotImplementedError("write your pallas kernel here")


# speedup vs baseline: 1.7153x; 1.7153x over previous
import functools

import jax
import jax.numpy as jnp
from jax import lax
from jax.experimental import pallas as pl
from jax.experimental.pallas import tpu as pltpu
from jax.experimental.pallas import tpu_sc as plsc

NC = 2
NS = 16
NW = NC * NS

IDX_W = 128
GATHERS_PER_CHUNK = 8
CHUNK = IDX_W * GATHERS_PER_CHUNK


def _sc_gather(table, idx):
    n = idx.shape[0]
    d = table.shape[1]
    per_w = n // NW
    n_chunks = per_w // CHUNK
    assert per_w % CHUNK == 0

    idx2d = idx.reshape(n // IDX_W, IDX_W)

    mesh = plsc.VectorSubcoreMesh(core_axis_name="c", subcore_axis_name="s")

    @functools.partial(
        pl.kernel,
        out_type=jax.ShapeDtypeStruct((n, d), jnp.float32),
        mesh=mesh,
        scratch_types=[
            pltpu.VMEM((GATHERS_PER_CHUNK, IDX_W), jnp.int32),
            pltpu.VMEM((CHUNK, d), jnp.float32),
            pltpu.SemaphoreType.DMA,
            pltpu.SemaphoreType.DMA,
        ],
        compiler_params=pltpu.CompilerParams(use_tc_tiling_on_sc=False),
    )
    def k(table_hbm, idx_hbm, out_hbm, idx_v, rows_v, gsem, osem):
        wid = lax.axis_index("s") * NC + lax.axis_index("c")
        row0 = wid * per_w
        irow0 = wid * (per_w // IDX_W)

        @pl.loop(0, n_chunks)
        def _(ci):
            base = row0 + ci * CHUNK
            ibase = irow0 + ci * GATHERS_PER_CHUNK
            pltpu.sync_copy(idx_hbm.at[pl.ds(ibase, GATHERS_PER_CHUNK)], idx_v)
            for g in range(GATHERS_PER_CHUNK):
                pltpu.async_copy(
                    table_hbm.at[idx_v.at[g]],
                    rows_v.at[pl.ds(g * IDX_W, IDX_W)],
                    gsem,
                )
            for g in range(GATHERS_PER_CHUNK):
                pltpu.make_async_copy(
                    table_hbm.at[idx_v.at[g]],
                    rows_v.at[pl.ds(g * IDX_W, IDX_W)],
                    gsem,
                ).wait()
            pltpu.async_copy(rows_v, out_hbm.at[pl.ds(base, CHUNK)], osem).wait()

    return k(table, idx2d)


def _add_body(a_ref, p_ref, o_ref):
    o_ref[...] = a_ref[...] + p_ref[0:1, :]


def _tc_add(tok2, pos8):
    m, k = tok2.shape
    blk = 256
    return pl.pallas_call(
        _add_body,
        grid=(m // blk,),
        in_specs=[
            pl.BlockSpec((blk, k), lambda i: (i, 0)),
            pl.BlockSpec((8, k), lambda i: (0, 0)),
        ],
        out_specs=pl.BlockSpec((blk, k), lambda i: (i, 0)),
        out_shape=jax.ShapeDtypeStruct((m, k), jnp.float32),
    )(tok2, pos8)


def kernel(x, token_table, pos_table):
    batch, seqlen = x.shape
    embed = token_table.shape[1]
    idx = x.reshape(-1).astype(jnp.int32)
    tok = _sc_gather(token_table, idx)
    tok2 = tok.reshape(batch, seqlen * embed)
    pos8 = jnp.tile(pos_table.reshape(1, seqlen * embed), (8, 1))
    out = _tc_add(tok2, pos8)
    return out.reshape(batch, seqlen, embed)


# speedup vs baseline: 2.0902x; 1.2185x over previous
import functools

import jax
import jax.numpy as jnp
from jax import lax
from jax.experimental import pallas as pl
from jax.experimental.pallas import tpu as pltpu
from jax.experimental.pallas import tpu_sc as plsc

NC = 2
NS = 16
NW = NC * NS

IDX_W = 128
GATHERS_PER_CHUNK = 8
CHUNK = IDX_W * GATHERS_PER_CHUNK


def _sc_gather(table, idx):
    n = idx.shape[0]
    d = table.shape[1]
    per_w = n // NW
    n_chunks = per_w // CHUNK
    assert per_w % CHUNK == 0

    idx2d = idx.reshape(n // IDX_W, IDX_W)

    mesh = plsc.VectorSubcoreMesh(core_axis_name="c", subcore_axis_name="s")

    @functools.partial(
        pl.kernel,
        out_type=jax.ShapeDtypeStruct((n, d), jnp.float32),
        mesh=mesh,
        scratch_types=[
            pltpu.VMEM((GATHERS_PER_CHUNK, IDX_W), jnp.int32),
            pltpu.VMEM((CHUNK, d), jnp.float32),
            pltpu.SemaphoreType.DMA,
            pltpu.SemaphoreType.DMA,
        ],
        compiler_params=pltpu.CompilerParams(use_tc_tiling_on_sc=False),
    )
    def k(table_hbm, idx_hbm, out_hbm, idx_v, rows_v, gsem, osem):
        wid = lax.axis_index("s") * NC + lax.axis_index("c")
        row0 = wid * per_w
        irow0 = wid * (per_w // IDX_W)

        @pl.loop(0, n_chunks)
        def _(ci):
            base = row0 + ci * CHUNK
            ibase = irow0 + ci * GATHERS_PER_CHUNK
            pltpu.sync_copy(idx_hbm.at[pl.ds(ibase, GATHERS_PER_CHUNK)], idx_v)
            for g in range(GATHERS_PER_CHUNK):
                pltpu.async_copy(
                    table_hbm.at[idx_v.at[g]],
                    rows_v.at[pl.ds(g * IDX_W, IDX_W)],
                    gsem,
                )
            for g in range(GATHERS_PER_CHUNK):
                pltpu.make_async_copy(
                    table_hbm.at[idx_v.at[g]],
                    rows_v.at[pl.ds(g * IDX_W, IDX_W)],
                    gsem,
                ).wait()
            pltpu.async_copy(rows_v, out_hbm.at[pl.ds(base, CHUNK)], osem).wait()

    return k(table, idx2d)


def _addt_body(tok_ref, pos_ref, o_ref):
    for r in range(50):
        t = tok_ref[0, pl.ds(128 * r, 128), :]
        t = t + pos_ref[pl.ds(r, 1), :]
        o_ref[pl.ds(128 * r, 128), :] = t.T


def _tc_add_transpose(tok3, pos2, nbatch):
    nblk = tok3.shape[0]
    return pl.pallas_call(
        _addt_body,
        grid=(nblk,),
        in_specs=[
            pl.BlockSpec((1, 6400, 128), lambda i: (i, 0, 0)),
            pl.BlockSpec((56, 128), lambda i: (0, 0)),
        ],
        out_specs=pl.BlockSpec((6400, 128), lambda i: (0, i)),
        out_shape=jax.ShapeDtypeStruct((6400, nbatch), jnp.float32),
    )(tok3, pos2)


def kernel(x, token_table, pos_table):
    batch, seqlen = x.shape
    embed = token_table.shape[1]
    nblk = batch // 128
    xi = x.astype(jnp.int32).reshape(nblk, 128, seqlen // 4, 4)
    idx = xi.transpose(0, 2, 1, 3).reshape(-1)
    tok = _sc_gather(token_table, idx)
    tok3 = tok.reshape(nblk, 6400, 128)
    pos2 = jnp.pad(pos_table.reshape(50, 128), ((0, 6), (0, 0)))
    out2 = _tc_add_transpose(tok3, pos2, batch)
    return out2.reshape(seqlen, embed, batch).transpose(2, 0, 1)


# speedup vs baseline: 3.1761x; 1.5196x over previous
import functools

import jax
import jax.numpy as jnp
from jax import lax
from jax.experimental import pallas as pl
from jax.experimental.pallas import tpu as pltpu
from jax.experimental.pallas import tpu_sc as plsc

NC = 2
NS = 16
NW = NC * NS

IDX_W = 128
GATHERS_PER_CHUNK = 8
CHUNK = IDX_W * GATHERS_PER_CHUNK


def _sc_gather(table, idx):
    n = idx.shape[0]
    d = table.shape[1]
    per_w = n // NW
    n_chunks = per_w // CHUNK
    assert per_w % CHUNK == 0

    idx2d = idx.reshape(n // IDX_W, IDX_W)

    mesh = plsc.VectorSubcoreMesh(core_axis_name="c", subcore_axis_name="s")

    @functools.partial(
        pl.kernel,
        out_type=jax.ShapeDtypeStruct((n, d), jnp.float32),
        mesh=mesh,
        scratch_types=[
            pltpu.VMEM((GATHERS_PER_CHUNK, IDX_W), jnp.int32),
            pltpu.VMEM((CHUNK, d), jnp.float32),
            pltpu.SemaphoreType.DMA,
            pltpu.SemaphoreType.DMA,
        ],
        compiler_params=pltpu.CompilerParams(use_tc_tiling_on_sc=False),
    )
    def k(table_hbm, idx_hbm, out_hbm, idx_v, rows_v, gsem, osem):
        wid = lax.axis_index("s") * NC + lax.axis_index("c")
        row0 = wid * per_w
        irow0 = wid * (per_w // IDX_W)

        @pl.loop(0, n_chunks)
        def _(ci):
            base = row0 + ci * CHUNK
            ibase = irow0 + ci * GATHERS_PER_CHUNK
            pltpu.sync_copy(idx_hbm.at[pl.ds(ibase, GATHERS_PER_CHUNK)], idx_v)
            for g in range(GATHERS_PER_CHUNK):
                pltpu.async_copy(
                    table_hbm.at[idx_v.at[g]],
                    rows_v.at[pl.ds(g * IDX_W, IDX_W)],
                    gsem,
                )
            for g in range(GATHERS_PER_CHUNK):
                pltpu.make_async_copy(
                    table_hbm.at[idx_v.at[g]],
                    rows_v.at[pl.ds(g * IDX_W, IDX_W)],
                    gsem,
                ).wait()
            pltpu.async_copy(rows_v, out_hbm.at[pl.ds(base, CHUNK)], osem).wait()

    return k(table, idx2d)


def _conv_body(tt_ref, o_ref):
    for t in range(16):
        for m in range(4):
            sub = tt_ref[:, pl.ds(512 * t + 128 * m, 128)]
            o_ref[pl.ds(128 * t, 128), pl.ds(32 * m, 32)] = sub.T


def _tc_table_convert(ttT, vpad):
    nv = ttT.shape[1]
    grid = (nv + 8191) // 8192
    return pl.pallas_call(
        _conv_body,
        grid=(grid,),
        in_specs=[pl.BlockSpec((32, 8192), lambda i: (0, i))],
        out_specs=pl.BlockSpec((2048, 128), lambda i: (i, 0)),
        out_shape=jax.ShapeDtypeStruct((vpad * 32 // 128, 128), jnp.float32),
    )(ttT)


def _addt_body(tok_ref, pos_ref, o_ref):
    for r in range(50):
        t = tok_ref[0, pl.ds(128 * r, 128), :]
        t = t + pos_ref[pl.ds(r, 1), :]
        o_ref[pl.ds(128 * r, 128), :] = t.T


def _tc_add_transpose(tok3, pos2, nbatch):
    nblk = tok3.shape[0]
    return pl.pallas_call(
        _addt_body,
        grid=(nblk,),
        in_specs=[
            pl.BlockSpec((1, 6400, 128), lambda i: (i, 0, 0)),
            pl.BlockSpec((56, 128), lambda i: (0, 0)),
        ],
        out_specs=pl.BlockSpec((6400, 128), lambda i: (0, i)),
        out_shape=jax.ShapeDtypeStruct((6400, nbatch), jnp.float32),
    )(tok3, pos2)


def kernel(x, token_table, pos_table):
    batch, seqlen = x.shape
    embed = token_table.shape[1]
    nblk = batch // 128
    xi = x.astype(jnp.int32).reshape(nblk, 128, seqlen // 4, 4)
    idx = xi.transpose(0, 2, 1, 3).reshape(-1)
    voc = token_table.shape[0]
    vpad = ((voc + 511) // 512) * 512
    tlin = _tc_table_convert(token_table.T, vpad)
    table = tlin.reshape(-1).reshape(vpad, embed)
    sidx = (idx // 512) * 512 + (idx % 128) * 4 + (idx % 512) // 128
    tok = _sc_gather(table, sidx)
    tok3 = tok.reshape(nblk, 6400, 128)
    pos2 = jnp.pad(pos_table.reshape(50, 128), ((0, 6), (0, 0)))
    out2 = _tc_add_transpose(tok3, pos2, batch)
    return out2.reshape(seqlen, embed, batch).transpose(2, 0, 1)
